# baseline (speedup 1.0000x reference)
import jax
import jax.numpy as jnp
from jax import lax
from jax.experimental import pallas as pl
from jax.experimental.pallas import tpu as pltpu
from jax.experimental.pallas import tpu_sc as plsc

_N = 10000
_E = 320000
_DIN = 128
_DHID = 128
_DOUT = 64

_NC = 2
_NS = 16
_NW = _NC * _NS
_EPW = _E // _NW
_K = 80
_NBLK = _EPW // _K
_NPAD = 10240
_RPT = _NPAD // _NS
_CW = 8

_BLK = 1000
_G = _N // _BLK


_SC_MESH = plsc.VectorSubcoreMesh(
    core_axis_name="c", subcore_axis_name="s",
    num_cores=_NC, num_subcores=_NS)
_SC_PARAMS = pltpu.CompilerParams(use_tc_tiling_on_sc=False)


def _make_sc_agg(D):
  out_type = [jax.ShapeDtypeStruct((_NC, _NPAD, D), jnp.float32)]
  scratch = [
      pltpu.VMEM((_NBLK, _K), jnp.int32),
      pltpu.VMEM((_NBLK, _K), jnp.int32),
      pltpu.VMEM((_K, D), jnp.float32),
      pltpu.VMEM((_K, D), jnp.float32),
      pltpu.VMEM_SHARED((_NPAD, D), jnp.float32),
      pltpu.SemaphoreType.DMA,
      pltpu.SemaphoreType.DMA,
      pltpu.SemaphoreType.DMA,
      pltpu.SemaphoreType.DMA,
  ]

  def body(u_hbm, src_hbm, dst_hbm, zrow_hbm, out_hbm,
           srci, dsti, rows0, rows1, accum, gsem0, gsem1, ssem0, ssem1):
    cid = lax.axis_index("c")
    sid = lax.axis_index("s")
    wid = cid * _NS + sid
    roff = pl.multiple_of(sid * _RPT, 8)
    icp0 = pltpu.async_copy(src_hbm.at[wid], srci, gsem0)
    icp1 = pltpu.async_copy(dst_hbm.at[wid], dsti, gsem1)
    pltpu.sync_copy(zrow_hbm.at[pl.ds(roff, _RPT)],
                    accum.at[pl.ds(roff, _RPT)])
    icp0.wait()
    icp1.wait()
    plsc.subcore_barrier()

    def wait_gather(j, rows, gsem):
      pltpu.make_async_copy(u_hbm.at[srci.at[j]], rows, gsem).wait()

    pltpu.async_copy(u_hbm.at[srci.at[0]], rows0, gsem0)
    pltpu.async_copy(u_hbm.at[srci.at[1]], rows1, gsem1)

    def step(jj, carry):
      j0 = 2 * jj
      j1 = j0 + 1
      wait_gather(j0, rows0, gsem0)
      d0 = pltpu.async_copy(rows0, accum.at[dsti.at[j0]], ssem0, add=True)
      wait_gather(j1, rows1, gsem1)
      d1 = pltpu.async_copy(rows1, accum.at[dsti.at[j1]], ssem1, add=True)
      d0.wait()
      pltpu.async_copy(u_hbm.at[srci.at[j0 + 2]], rows0, gsem0)
      d1.wait()
      pltpu.async_copy(u_hbm.at[srci.at[j1 + 2]], rows1, gsem1)
      return carry

    lax.fori_loop(0, (_NBLK - 3) // 2, step, 0)
    ja, jb, jc = _NBLK - 3, _NBLK - 2, _NBLK - 1
    wait_gather(ja, rows0, gsem0)
    da = pltpu.async_copy(rows0, accum.at[dsti.at[ja]], ssem0, add=True)
    wait_gather(jb, rows1, gsem1)
    db = pltpu.async_copy(rows1, accum.at[dsti.at[jb]], ssem1, add=True)
    da.wait()
    pltpu.async_copy(u_hbm.at[srci.at[jc]], rows0, gsem0)
    db.wait()
    wait_gather(jc, rows0, gsem0)
    pltpu.sync_copy(rows0, accum.at[dsti.at[jc]], add=True)
    plsc.subcore_barrier()
    pltpu.sync_copy(accum.at[pl.ds(roff, _RPT)],
                    out_hbm.at[cid, pl.ds(roff, _RPT)])

  return pl.kernel(body, out_type=out_type, mesh=_SC_MESH,
                   scratch_types=scratch, compiler_params=_SC_PARAMS)


def _make_sc_cnt():
  out_type = [jax.ShapeDtypeStruct((_NC, _NPAD, _CW), jnp.float32)]
  scratch = [
      pltpu.VMEM((_NBLK, _K), jnp.int32),
      pltpu.VMEM((_K, _CW), jnp.float32),
      pltpu.VMEM_SHARED((_NPAD, _CW), jnp.float32),
      pltpu.SemaphoreType.DMA,
  ]

  def body(dst_hbm, zc_hbm, ones_hbm, cnt_hbm, dsti, ones, cacc, sem):
    cid = lax.axis_index("c")
    sid = lax.axis_index("s")
    wid = cid * _NS + sid
    roff = pl.multiple_of(sid * _RPT, 8)
    icp = pltpu.async_copy(dst_hbm.at[wid], dsti, sem)
    pltpu.sync_copy(zc_hbm.at[pl.ds(roff, _RPT)],
                    cacc.at[pl.ds(roff, _RPT)])
    pltpu.sync_copy(ones_hbm, ones)
    icp.wait()
    plsc.subcore_barrier()

    def step(j, carry):
      pltpu.sync_copy(ones, cacc.at[dsti.at[j]], add=True)
      return carry

    lax.fori_loop(0, _NBLK, step, 0)
    plsc.subcore_barrier()
    pltpu.sync_copy(cacc.at[pl.ds(roff, _RPT)],
                    cnt_hbm.at[cid, pl.ds(roff, _RPT)])

  return pl.kernel(body, out_type=out_type, mesh=_SC_MESH,
                   scratch_types=scratch, compiler_params=_SC_PARAMS)


_sc_agg0 = _make_sc_agg(_DHID)
_sc_agg1 = _make_sc_agg(_DOUT)
_sc_cnt = _make_sc_cnt()


def _tc_mm(x, W):
  din = x.shape[1]
  dout = W.shape[1]

  def body(x_ref, w_ref, o_ref):
    o_ref[...] = jnp.dot(x_ref[...], w_ref[...],
                         preferred_element_type=jnp.float32)

  return pl.pallas_call(
      body,
      grid=(_G,),
      in_specs=[
          pl.BlockSpec((_BLK, din), lambda i: (i, 0)),
          pl.BlockSpec((din, dout), lambda i: (0, 0)),
      ],
      out_specs=pl.BlockSpec((_BLK, dout), lambda i: (i, 0)),
      out_shape=jax.ShapeDtypeStruct((_N, dout), jnp.float32),
  )(x, W)


def _tc_mid(S0, C0, xr0, b0, W_l1):
  def body(s_ref, c_ref, xr, b, wl, u1, h_ref):
    s = s_ref[0] + s_ref[1]
    c = c_ref[0][:, 0:1] + c_ref[1][:, 0:1]
    inv = 1.0 / jnp.maximum(c, 1.0)
    h = jnp.maximum(s * inv + xr[...] + b[...], 0.0)
    u1[...] = jnp.dot(h, wl[...], preferred_element_type=jnp.float32)
    h_ref[...] = h

  return pl.pallas_call(
      body,
      grid=(_G,),
      in_specs=[
          pl.BlockSpec((_NC, _BLK, _DHID), lambda i: (0, i, 0)),
          pl.BlockSpec((_NC, _BLK, _CW), lambda i: (0, i, 0)),
          pl.BlockSpec((_BLK, _DHID), lambda i: (i, 0)),
          pl.BlockSpec((1, _DHID), lambda i: (0, 0)),
          pl.BlockSpec((_DHID, _DOUT), lambda i: (0, 0)),
      ],
      out_specs=[pl.BlockSpec((_BLK, _DOUT), lambda i: (i, 0)),
                 pl.BlockSpec((_BLK, _DHID), lambda i: (i, 0))],
      out_shape=[jax.ShapeDtypeStruct((_N, _DOUT), jnp.float32),
                 jax.ShapeDtypeStruct((_N, _DHID), jnp.float32)],
  )(S0, C0, xr0, b0, W_l1)


def _tc_out(S1, C0, hr1, b1):
  def body(s_ref, c_ref, hr, b, o_ref):
    s = s_ref[0] + s_ref[1]
    c = c_ref[0][:, 0:1] + c_ref[1][:, 0:1]
    inv = 1.0 / jnp.maximum(c, 1.0)
    z = s * inv + hr[...] + b[...]
    m = jnp.max(z, axis=1, keepdims=True)
    e = jnp.exp(z - m)
    o_ref[...] = (z - m) - jnp.log(jnp.sum(e, axis=1, keepdims=True))

  return pl.pallas_call(
      body,
      grid=(_G,),
      in_specs=[
          pl.BlockSpec((_NC, _BLK, _DOUT), lambda i: (0, i, 0)),
          pl.BlockSpec((_NC, _BLK, _CW), lambda i: (0, i, 0)),
          pl.BlockSpec((_BLK, _DOUT), lambda i: (i, 0)),
          pl.BlockSpec((1, _DOUT), lambda i: (0, 0)),
      ],
      out_specs=pl.BlockSpec((_BLK, _DOUT), lambda i: (i, 0)),
      out_shape=jax.ShapeDtypeStruct((_N, _DOUT), jnp.float32),
  )(S1, C0, hr1, b1)


def kernel(x, adj, default_chunk_size, chunk_sizes_diff,
           W_l0, b_l0, W_r0, W_l1, b_l1, W_r1):
  del default_chunk_size, chunk_sizes_diff
  src = adj[0].reshape(_NW, _NBLK, _K)
  dst = adj[1].reshape(_NW, _NBLK, _K)
  zc = jnp.zeros((_NPAD, _CW), jnp.float32)
  ones_h = jnp.ones((_K, _CW), jnp.float32)
  cnt_out = _sc_cnt(dst, zc, ones_h)
  C0 = cnt_out[0] if isinstance(cnt_out, (list, tuple)) else cnt_out
  u0 = _tc_mm(x, W_l0)
  zrow0 = jnp.zeros((_NPAD, _DHID), jnp.float32)
  out0 = _sc_agg0(u0, src, dst, zrow0)
  S0 = out0[0] if isinstance(out0, (list, tuple)) else out0
  xr0 = _tc_mm(x, W_r0)
  b0 = b_l0.reshape(1, _DHID)
  b1 = b_l1.reshape(1, _DOUT)
  u1, h = _tc_mid(S0, C0, xr0, b0, W_l1)
  zrow1 = jnp.zeros((_NPAD, _DOUT), jnp.float32)
  out1 = _sc_agg1(u1, src, dst, zrow1)
  S1 = out1[0] if isinstance(out1, (list, tuple)) else out1
  hr1 = _tc_mm(h, W_r1)
  return _tc_out(S1, C0, hr1, b1)


# speedup vs baseline: 1.1754x; 1.1754x over previous
import jax
import jax.numpy as jnp
from jax import lax
from jax.experimental import pallas as pl
from jax.experimental.pallas import tpu as pltpu
from jax.experimental.pallas import tpu_sc as plsc

_N = 10000
_E = 320000
_DIN = 128
_DHID = 128
_DOUT = 64

_NC = 2
_NS = 16
_NW = _NC * _NS
_EPW = _E // _NW
_K = 80
_NBLK = _EPW // _K
_NPAD = 10240
_RPT = _NPAD // _NS
_CW = 8

_BLK = 1000
_G = _N // _BLK


_SC_MESH = plsc.VectorSubcoreMesh(
    core_axis_name="c", subcore_axis_name="s",
    num_cores=_NC, num_subcores=_NS)
_SC_PARAMS = pltpu.CompilerParams(use_tc_tiling_on_sc=False)


def _make_sc_agg(D, stage_src=False):
  out_type = [jax.ShapeDtypeStruct((_NC, _NPAD, D), jnp.float32)]
  scratch = [
      pltpu.VMEM((_NBLK, _K), jnp.int32),
      pltpu.VMEM((_NBLK, _K), jnp.int32),
      pltpu.VMEM((_K, D), jnp.float32),
      pltpu.VMEM((_K, D), jnp.float32),
      pltpu.VMEM_SHARED((_NPAD, D), jnp.float32),
      pltpu.SemaphoreType.DMA,
      pltpu.SemaphoreType.DMA,
  ]
  if stage_src:
    scratch.append(pltpu.VMEM_SHARED((_N, D), jnp.float32))

  def body(u_hbm, src_hbm, dst_hbm, zrow_hbm, out_hbm,
           srci, dsti, rows0, rows1, accum, gsem0, gsem1, *maybe_stage):
    cid = lax.axis_index("c")
    sid = lax.axis_index("s")
    wid = cid * _NS + sid
    roff = pl.multiple_of(sid * _RPT, 8)
    icp0 = pltpu.async_copy(src_hbm.at[wid], srci, gsem0)
    icp1 = pltpu.async_copy(dst_hbm.at[wid], dsti, gsem1)
    pltpu.sync_copy(zrow_hbm.at[pl.ds(roff, _RPT)],
                    accum.at[pl.ds(roff, _RPT)])
    if stage_src:
      u_tab = maybe_stage[0]
      soff = pl.multiple_of(sid * 624, 8)
      pltpu.sync_copy(u_hbm.at[pl.ds(soff, 624)],
                      u_tab.at[pl.ds(soff, 624)])
      @pl.when(sid == 0)
      def _():
        pltpu.sync_copy(u_hbm.at[pl.ds(9984, _N - 9984)],
                        u_tab.at[pl.ds(9984, _N - 9984)])
    else:
      u_tab = u_hbm
    icp0.wait()
    icp1.wait()
    plsc.subcore_barrier()

    def wait_gather(j, rows, gsem):
      pltpu.make_async_copy(u_tab.at[srci.at[j]], rows, gsem).wait()

    pltpu.async_copy(u_tab.at[srci.at[0]], rows0, gsem0)

    def step(jj, carry):
      j0 = 2 * jj
      j1 = j0 + 1
      pltpu.async_copy(u_tab.at[srci.at[j1]], rows1, gsem1)
      wait_gather(j0, rows0, gsem0)
      pltpu.sync_copy(rows0, accum.at[dsti.at[j0]], add=True)
      pltpu.async_copy(u_tab.at[srci.at[j0 + 2]], rows0, gsem0)
      wait_gather(j1, rows1, gsem1)
      pltpu.sync_copy(rows1, accum.at[dsti.at[j1]], add=True)
      return carry

    lax.fori_loop(0, (_NBLK - 1) // 2, step, 0)
    j_last = _NBLK - 1
    wait_gather(j_last, rows0, gsem0)
    pltpu.sync_copy(rows0, accum.at[dsti.at[j_last]], add=True)
    plsc.subcore_barrier()
    pltpu.sync_copy(accum.at[pl.ds(roff, _RPT)],
                    out_hbm.at[cid, pl.ds(roff, _RPT)])

  return pl.kernel(body, out_type=out_type, mesh=_SC_MESH,
                   scratch_types=scratch, compiler_params=_SC_PARAMS)


def _make_sc_cnt():
  out_type = [jax.ShapeDtypeStruct((_NC, _NPAD, _CW), jnp.float32)]
  scratch = [
      pltpu.VMEM((_NBLK, _K), jnp.int32),
      pltpu.VMEM((_K, _CW), jnp.float32),
      pltpu.VMEM_SHARED((_NPAD, _CW), jnp.float32),
      pltpu.SemaphoreType.DMA,
  ]

  def body(dst_hbm, zc_hbm, ones_hbm, cnt_hbm, dsti, ones, cacc, sem):
    cid = lax.axis_index("c")
    sid = lax.axis_index("s")
    wid = cid * _NS + sid
    roff = pl.multiple_of(sid * _RPT, 8)
    icp = pltpu.async_copy(dst_hbm.at[wid], dsti, sem)
    pltpu.sync_copy(zc_hbm.at[pl.ds(roff, _RPT)],
                    cacc.at[pl.ds(roff, _RPT)])
    pltpu.sync_copy(ones_hbm, ones)
    icp.wait()
    plsc.subcore_barrier()

    def step(j, carry):
      pltpu.sync_copy(ones, cacc.at[dsti.at[j]], add=True)
      return carry

    lax.fori_loop(0, _NBLK, step, 0)
    plsc.subcore_barrier()
    pltpu.sync_copy(cacc.at[pl.ds(roff, _RPT)],
                    cnt_hbm.at[cid, pl.ds(roff, _RPT)])

  return pl.kernel(body, out_type=out_type, mesh=_SC_MESH,
                   scratch_types=scratch, compiler_params=_SC_PARAMS)


_sc_agg0 = _make_sc_agg(_DHID)
_sc_agg1 = _make_sc_agg(_DOUT, stage_src=True)
_sc_cnt = _make_sc_cnt()


def _tc_mm(x, W):
  din = x.shape[1]
  dout = W.shape[1]

  def body(x_ref, w_ref, o_ref):
    o_ref[...] = jnp.dot(x_ref[...], w_ref[...],
                         preferred_element_type=jnp.float32)

  return pl.pallas_call(
      body,
      grid=(_G,),
      in_specs=[
          pl.BlockSpec((_BLK, din), lambda i: (i, 0)),
          pl.BlockSpec((din, dout), lambda i: (0, 0)),
      ],
      out_specs=pl.BlockSpec((_BLK, dout), lambda i: (i, 0)),
      out_shape=jax.ShapeDtypeStruct((_N, dout), jnp.float32),
  )(x, W)


def _tc_mid(S0, C0, xr0, b0, W_l1):
  def body(s_ref, c_ref, xr, b, wl, u1, h_ref):
    s = s_ref[0] + s_ref[1]
    c = c_ref[0][:, 0:1] + c_ref[1][:, 0:1]
    inv = 1.0 / jnp.maximum(c, 1.0)
    h = jnp.maximum(s * inv + xr[...] + b[...], 0.0)
    u1[...] = jnp.dot(h, wl[...], preferred_element_type=jnp.float32)
    h_ref[...] = h

  return pl.pallas_call(
      body,
      grid=(_G,),
      in_specs=[
          pl.BlockSpec((_NC, _BLK, _DHID), lambda i: (0, i, 0)),
          pl.BlockSpec((_NC, _BLK, _CW), lambda i: (0, i, 0)),
          pl.BlockSpec((_BLK, _DHID), lambda i: (i, 0)),
          pl.BlockSpec((1, _DHID), lambda i: (0, 0)),
          pl.BlockSpec((_DHID, _DOUT), lambda i: (0, 0)),
      ],
      out_specs=[pl.BlockSpec((_BLK, _DOUT), lambda i: (i, 0)),
                 pl.BlockSpec((_BLK, _DHID), lambda i: (i, 0))],
      out_shape=[jax.ShapeDtypeStruct((_N, _DOUT), jnp.float32),
                 jax.ShapeDtypeStruct((_N, _DHID), jnp.float32)],
  )(S0, C0, xr0, b0, W_l1)


def _tc_out(S1, C0, hr1, b1):
  def body(s_ref, c_ref, hr, b, o_ref):
    s = s_ref[0] + s_ref[1]
    c = c_ref[0][:, 0:1] + c_ref[1][:, 0:1]
    inv = 1.0 / jnp.maximum(c, 1.0)
    z = s * inv + hr[...] + b[...]
    m = jnp.max(z, axis=1, keepdims=True)
    e = jnp.exp(z - m)
    o_ref[...] = (z - m) - jnp.log(jnp.sum(e, axis=1, keepdims=True))

  return pl.pallas_call(
      body,
      grid=(_G,),
      in_specs=[
          pl.BlockSpec((_NC, _BLK, _DOUT), lambda i: (0, i, 0)),
          pl.BlockSpec((_NC, _BLK, _CW), lambda i: (0, i, 0)),
          pl.BlockSpec((_BLK, _DOUT), lambda i: (i, 0)),
          pl.BlockSpec((1, _DOUT), lambda i: (0, 0)),
      ],
      out_specs=pl.BlockSpec((_BLK, _DOUT), lambda i: (i, 0)),
      out_shape=jax.ShapeDtypeStruct((_N, _DOUT), jnp.float32),
  )(S1, C0, hr1, b1)


def kernel(x, adj, default_chunk_size, chunk_sizes_diff,
           W_l0, b_l0, W_r0, W_l1, b_l1, W_r1):
  del default_chunk_size, chunk_sizes_diff
  src = adj[0].reshape(_NW, _NBLK, _K)
  dst = adj[1].reshape(_NW, _NBLK, _K)
  zc = jnp.zeros((_NPAD, _CW), jnp.float32)
  ones_h = jnp.ones((_K, _CW), jnp.float32)
  cnt_out = _sc_cnt(dst, zc, ones_h)
  C0 = cnt_out[0] if isinstance(cnt_out, (list, tuple)) else cnt_out
  u0 = _tc_mm(x, W_l0)
  zrow0 = jnp.zeros((_NPAD, _DHID), jnp.float32)
  out0 = _sc_agg0(u0, src, dst, zrow0)
  S0 = out0[0] if isinstance(out0, (list, tuple)) else out0
  xr0 = _tc_mm(x, W_r0)
  b0 = b_l0.reshape(1, _DHID)
  b1 = b_l1.reshape(1, _DOUT)
  u1, h = _tc_mid(S0, C0, xr0, b0, W_l1)
  zrow1 = jnp.zeros((_NPAD, _DOUT), jnp.float32)
  out1 = _sc_agg1(u1, src, dst, zrow1)
  S1 = out1[0] if isinstance(out1, (list, tuple)) else out1
  hr1 = _tc_mm(h, W_r1)
  return _tc_out(S1, C0, hr1, b1)
